# baseline (speedup 1.0000x reference)
import jax
import jax.numpy as jnp
from jax import lax
from jax.experimental import pallas as pl
from jax.experimental.pallas import tpu as pltpu
from jax.experimental.pallas import tpu_sc as plsc

L = 4
H = 128
N = 10000
E = 320000
R = 1000
S2 = 40000
DE = 16
ZV = 100

NC = 2
NS = 16
NW = NC * NS

NP = 10240
EP = 327680
RP = 1024
C = 32
GC = EP // C
PH = 160
NPW = NP // NW
NPT = NP // NS

_f32 = jnp.float32
_i32 = jnp.int32


def _ln(v, scale, bias):
    mu = jnp.mean(v, axis=-1, keepdims=True)
    d = v - mu
    var = jnp.mean(d * d, axis=-1, keepdims=True)
    return d * lax.rsqrt(var + 1e-5) * scale + bias


def _sc_prep(f2s_hbm, s2t_hbm, nidx_hbm, root_hbm, z0_hbm, z0t_hbm,
             d2_hbm, ri_hbm, z0e_hbm,
             s2t_v, root_v, idx_v, out_v, zi_v, zr_v, sem):
    cid = lax.axis_index("c")
    sid = lax.axis_index("s")
    wid = sid * NC + cid

    pltpu.sync_copy(s2t_hbm, s2t_v)
    pltpu.sync_copy(root_hbm, root_v)

    def d2_chunk(i, _):
        base = wid * (EP // NW) + i * 1024

        pltpu.sync_copy(f2s_hbm.at[pl.ds(base, 1024)], idx_v)

        def gat(j, _):
            g = plsc.load_gather(s2t_v, [idx_v[pl.ds(j * 16, 16)]])
            out_v[pl.ds(j * 16, 16)] = g
            return 0

        lax.fori_loop(0, 64, gat, 0)
        pltpu.sync_copy(out_v, d2_hbm.at[pl.ds(base, 1024)])
        return 0

    lax.fori_loop(0, EP // NW // 1024, d2_chunk, 0)

    nbase = wid * NPW
    pltpu.sync_copy(nidx_hbm.at[pl.ds(nbase, NPW)], idx_v.at[pl.ds(0, NPW)])

    def gat_r(j, _):
        g = plsc.load_gather(root_v, [idx_v[pl.ds(j * 16, 16)]])
        out_v[pl.ds(j * 16, 16)] = g
        return 0

    lax.fori_loop(0, NPW // 16, gat_r, 0)
    pltpu.sync_copy(out_v.at[pl.ds(0, NPW)], ri_hbm.at[pl.ds(nbase, NPW)])

    def z0_chunk(j, _):
        b0 = nbase + j * 80
        pltpu.sync_copy(z0_hbm.at[pl.ds(b0, 80)], zi_v)
        pltpu.async_copy(z0t_hbm.at[zi_v], zr_v, sem).wait()
        pltpu.sync_copy(zr_v, z0e_hbm.at[pl.ds(b0, 80)])
        return 0

    lax.fori_loop(0, NPW // 80, z0_chunk, 0)


def _run_sc_prep(f2s_p, s2t, nidx_p, root_p, z0_p, z0t_p):
    mesh = plsc.VectorSubcoreMesh(core_axis_name="c", subcore_axis_name="s")
    fn = pl.kernel(
        _sc_prep,
        out_type=[
            jax.ShapeDtypeStruct((EP,), _i32),
            jax.ShapeDtypeStruct((NP,), _i32),
            jax.ShapeDtypeStruct((NP, H), _f32),
        ],
        mesh=mesh,
        compiler_params=pltpu.CompilerParams(needs_layout_passes=False),
        scratch_types=[
            pltpu.VMEM((S2,), _i32),
            pltpu.VMEM((RP,), _i32),
            pltpu.VMEM((1024,), _i32),
            pltpu.VMEM((1024,), _i32),
            pltpu.VMEM((80,), _i32),
            pltpu.VMEM((80, H), _f32),
            pltpu.SemaphoreType.DMA,
        ],
    )
    return fn(f2s_p, s2t, nidx_p, root_p, z0_p, z0t_p)


def _sc_edge(h_hbm, emb_hbm, src_hbm, dst_hbm, d2_hbm, ri_hbm,
             part_hbm, hg_hbm,
             srcf, dstf, d2f, idx30, idx31, hgi_v, rows0, rows1, emb0, emb1,
             m0, m1, semd0, semd1, sems0, sems1, semz, acc):
    cid = lax.axis_index("c")
    sid = lax.axis_index("s")
    wid = sid * NC + cid

    rows = (rows0, rows1)
    embv = (emb0, emb1)
    mv = (m0, m1)
    idx3 = (idx30, idx31)
    semd = (semd0, semd1)
    sems = (sems0, sems1)

    def zrow(r, _):
        for k in range(H // 16):
            m0[r, pl.ds(k * 16, 16)] = jnp.zeros((16,), _f32)
        return 0

    lax.fori_loop(0, C, zrow, 0)
    for b in range(NPT // C):
        pltpu.async_copy(m0, acc.at[pl.ds(sid * NPT + b * C, C)], semz)
    for b in range(NPT // C):
        pltpu.make_async_copy(m0, acc.at[pl.ds(0, C)], semz).wait()

    def hg_chunk2(j, _):
        b0 = wid * NPW + j * C
        pltpu.sync_copy(ri_hbm.at[pl.ds(b0, C)], hgi_v)
        pltpu.async_copy(h_hbm.at[hgi_v], emb0, semz).wait()
        pltpu.sync_copy(emb0, hg_hbm.at[pl.ds(b0, C)])
        return 0

    lax.fori_loop(0, NPW // C, hg_chunk2, 0)

    plsc.subcore_barrier()

    def run_phase(cbase):
        ebase = cbase * C

        pltpu.sync_copy(src_hbm.at[pl.ds(ebase, PH * C)], srcf)
        pltpu.sync_copy(dst_hbm.at[pl.ds(ebase, PH * C)], dstf)
        pltpu.sync_copy(d2_hbm.at[pl.ds(ebase, PH * C)], d2f)

        def start_dat(c, s):
            pltpu.async_copy(h_hbm.at[srcf.at[pl.ds(c * C, C)]],
                             rows[s], semd[s])
            pltpu.async_copy(emb_hbm.at[pl.ds(ebase + c * C, C)],
                             embv[s], semd[s])

        def wait_dat(c, s):
            pltpu.make_async_copy(h_hbm.at[srcf.at[pl.ds(c * C, C)]],
                                  rows[s], semd[s]).wait()
            pltpu.make_async_copy(emb_hbm.at[pl.ds(ebase + c * C, C)],
                                  embv[s], semd[s]).wait()

        def stage_idx3(c, s):
            for k in range(C // 16):
                idx3[s][0, pl.ds(k * 16, 16)] = dstf[pl.ds(c * C + k * 16, 16)]
                idx3[s][1, pl.ds(k * 16, 16)] = d2f[pl.ds(c * C + k * 16, 16)]

        def compute(s):
            r_s, e_s, m_s = rows[s], embv[s], mv[s]

            def row(r, _):
                for u in range(2):
                    for k in range(H // 16):
                        sl = pl.ds(k * 16, 16)
                        m_s[2 * r + u, sl] = jnp.maximum(
                            r_s[2 * r + u, sl] + e_s[2 * r + u, sl], 0.0)
                return 0

            lax.fori_loop(0, C // 2, row, 0)

        def start_scat(s):
            pltpu.async_copy(mv[s], acc.at[idx3[s].at[0]], sems[s],
                             add=True)
            pltpu.async_copy(mv[s], acc.at[idx3[s].at[1]], sems[s],
                             add=True)

        def wait_scat(s):
            pltpu.make_async_copy(mv[s], acc.at[idx3[s].at[0]],
                                  sems[s]).wait()
            pltpu.make_async_copy(mv[s], acc.at[idx3[s].at[1]],
                                  sems[s]).wait()

        start_dat(0, 0)
        start_dat(1, 1)
        for s in range(2):
            wait_dat(s, s)
            stage_idx3(s, s)
            compute(s)
            start_scat(s)
            start_dat(s + 2, s)

        def pair(j, _):
            for s in range(2):
                c = 2 * j + s
                wait_dat(c, s)
                wait_scat(s)
                stage_idx3(c, s)
                compute(s)
                start_scat(s)
                start_dat(c + 2, s)
            return 0

        lax.fori_loop(1, PH // 2 - 1, pair, 0)

        for s in range(2):
            c = PH - 2 + s
            wait_dat(c, s)
            wait_scat(s)
            stage_idx3(c, s)
            compute(s)
            start_scat(s)
        wait_scat(0)
        wait_scat(1)

    base0 = wid * (GC // NW)
    for p in range(GC // NW // PH):
        run_phase(base0 + p * PH)

    plsc.subcore_barrier()
    pltpu.sync_copy(acc.at[pl.ds(sid * NPT, NPT)],
                    part_hbm.at[cid, pl.ds(sid * NPT, NPT)])


def _run_sc_edge(h, emb, src_p, dst_p, d2, ri):
    mesh = plsc.VectorSubcoreMesh(core_axis_name="c", subcore_axis_name="s")
    fn = pl.kernel(
        _sc_edge,
        out_type=[
            jax.ShapeDtypeStruct((NC, NP, H), _f32),
            jax.ShapeDtypeStruct((NP, H), _f32),
        ],
        mesh=mesh,
        compiler_params=pltpu.CompilerParams(internal_scratch_in_bytes=4096),
        scratch_types=[
            pltpu.VMEM((PH * C,), _i32),
            pltpu.VMEM((PH * C,), _i32),
            pltpu.VMEM((PH * C,), _i32),
            pltpu.VMEM((2, C), _i32),
            pltpu.VMEM((2, C), _i32),
            pltpu.VMEM((C,), _i32),
            pltpu.VMEM((C, H), _f32),
            pltpu.VMEM((C, H), _f32),
            pltpu.VMEM((C, H), _f32),
            pltpu.VMEM((C, H), _f32),
            pltpu.VMEM((C, H), _f32),
            pltpu.VMEM((C, H), _f32),
            pltpu.SemaphoreType.DMA,
            pltpu.SemaphoreType.DMA,
            pltpu.SemaphoreType.DMA,
            pltpu.SemaphoreType.DMA,
            pltpu.SemaphoreType.DMA,
            pltpu.VMEM_SHARED((NP, H), _f32),
        ],
    )
    return fn(h, emb, src_p, dst_p, d2, ri)


def _sc_pool(h_hbm, nidx_hbm, part_hbm, idx_v, rows_v, acc):
    cid = lax.axis_index("c")
    sid = lax.axis_index("s")
    wid = sid * NC + cid

    def zrow(r, _):
        for k in range(H // 16):
            rows_v[r, pl.ds(k * 16, 16)] = jnp.zeros((16,), _f32)
        return 0

    lax.fori_loop(0, 80, zrow, 0)
    rpt = RP // NS
    pltpu.sync_copy(rows_v.at[pl.ds(0, rpt)],
                    acc.at[pl.ds(sid * rpt, rpt)])
    plsc.subcore_barrier()

    def chunk(j, _):
        b0 = wid * NPW + j * 80
        pltpu.sync_copy(nidx_hbm.at[pl.ds(b0, 80)], idx_v)
        pltpu.sync_copy(h_hbm.at[pl.ds(b0, 80)], rows_v)
        pltpu.sync_copy(rows_v, acc.at[idx_v], add=True)
        return 0

    lax.fori_loop(0, NPW // 80, chunk, 0)

    plsc.subcore_barrier()
    pltpu.sync_copy(acc.at[pl.ds(sid * rpt, rpt)],
                    part_hbm.at[cid, pl.ds(sid * rpt, rpt)])


def _run_sc_pool(h, nidx_p):
    mesh = plsc.VectorSubcoreMesh(core_axis_name="c", subcore_axis_name="s")
    fn = pl.kernel(
        _sc_pool,
        out_type=jax.ShapeDtypeStruct((NC, RP, H), _f32),
        mesh=mesh,
        scratch_types=[
            pltpu.VMEM((80,), _i32),
            pltpu.VMEM((80, H), _f32),
            pltpu.VMEM_SHARED((RP, H), _f32),
        ],
    )
    return fn(h, nidx_p)


_BN = 1024


def _tc_init_body(x_ref, z0e_ref, Wi_ref, bi_ref, o_ref):
    pid = pl.program_id(0)
    v = jnp.dot(x_ref[...], Wi_ref[...], preferred_element_type=_f32)
    v = v + bi_ref[...] + z0e_ref[...]
    rows = pid * _BN + lax.broadcasted_iota(_i32, (_BN, H), 0)
    o_ref[...] = jnp.where(rows < N, v, 0.0)


def _run_tc_init(x_p, z0e, W_init, b_init):
    return pl.pallas_call(
        _tc_init_body,
        grid=(NP // _BN,),
        in_specs=[
            pl.BlockSpec((_BN, H), lambda i: (i, 0)),
            pl.BlockSpec((_BN, H), lambda i: (i, 0)),
            pl.BlockSpec((H, H), lambda i: (0, 0)),
            pl.BlockSpec((1, H), lambda i: (0, 0)),
        ],
        out_specs=pl.BlockSpec((_BN, H), lambda i: (i, 0)),
        out_shape=jax.ShapeDtypeStruct((NP, H), _f32),
    )(x_p, z0e, W_init, b_init.reshape(1, H))


_BE = 1280


def _tc_emb_body(ea_ref, We_ref, be_ref, o_ref):
    pid = pl.program_id(0)
    v = jnp.dot(ea_ref[...], We_ref[...], preferred_element_type=_f32)
    v = v + be_ref[...]
    rows = pid * _BE + lax.broadcasted_iota(_i32, (_BE, H), 0)
    o_ref[...] = jnp.where(rows < E, v, -1e30)


def _run_tc_emb(ea, We_l, be_l):
    last = E // _BE - 1
    return pl.pallas_call(
        _tc_emb_body,
        grid=(EP // _BE,),
        in_specs=[
            pl.BlockSpec((_BE, DE), lambda i: (jnp.minimum(i, last), 0)),
            pl.BlockSpec((DE, H), lambda i: (0, 0)),
            pl.BlockSpec((1, H), lambda i: (0, 0)),
        ],
        out_specs=pl.BlockSpec((_BE, H), lambda i: (i, 0)),
        out_shape=jax.ShapeDtypeStruct((EP, H), _f32),
    )(ea, We_l, be_l.reshape(1, H))


def _tc_layer_body(p0_ref, p1_ref, h_ref, hg_ref, Wg_ref, bg_ref, ns_ref,
                   nb_ref, Wr_ref, br_ref, rs_ref, rb_ref, o_ref):
    pid = pl.program_id(0)
    agg = p0_ref[0] + p1_ref[0]
    out = jnp.dot(agg, Wg_ref[...], preferred_element_type=_f32) + bg_ref[...]
    out = _ln(out, ns_ref[...], nb_ref[...])
    hr = h_ref[...] + hg_ref[...]
    hr = jnp.dot(hr, Wr_ref[...], preferred_element_type=_f32) + br_ref[...]
    hr = jnp.maximum(hr, 0.0)
    hr = _ln(hr, rs_ref[...], rb_ref[...])
    v = out + hr
    rows = pid * _BN + lax.broadcasted_iota(_i32, (_BN, H), 0)
    o_ref[...] = jnp.where(rows < N, v, 0.0)


def _run_tc_layer(part, h, hg, Wg_l, bg_l, ns_l, nb_l, Wr_l, br_l, rs_l,
                  rb_l):
    row_spec = pl.BlockSpec((_BN, H), lambda i: (i, 0))
    p0 = pl.BlockSpec((1, _BN, H), lambda i: (0, i, 0))
    p1 = pl.BlockSpec((1, _BN, H), lambda i: (1, i, 0))
    mat_spec = pl.BlockSpec((H, H), lambda i: (0, 0))
    vec_spec = pl.BlockSpec((1, H), lambda i: (0, 0))
    return pl.pallas_call(
        _tc_layer_body,
        grid=(NP // _BN,),
        in_specs=[p0, p1, row_spec, row_spec,
                  mat_spec, vec_spec, vec_spec, vec_spec,
                  mat_spec, vec_spec, vec_spec, vec_spec],
        out_specs=row_spec,
        out_shape=jax.ShapeDtypeStruct((NP, H), _f32),
    )(part, part, h, hg, Wg_l, bg_l.reshape(1, H), ns_l.reshape(1, H),
      nb_l.reshape(1, H), Wr_l, br_l.reshape(1, H), rs_l.reshape(1, H),
      rb_l.reshape(1, H))


_BR = 200


def _tc_final_body(q0_ref, q1_ref, Wj_ref, bj_ref, o_ref):
    q = q0_ref[0] + q1_ref[0]
    o_ref[...] = jnp.dot(q, Wj_ref[...], preferred_element_type=_f32) + bj_ref[...]


def _run_tc_final(qpart, W_jk, b_jk):
    return pl.pallas_call(
        _tc_final_body,
        grid=(R // _BR,),
        in_specs=[pl.BlockSpec((1, _BR, H), lambda i: (0, i, 0)),
                  pl.BlockSpec((1, _BR, H), lambda i: (1, i, 0)),
                  pl.BlockSpec((H, H), lambda i: (0, 0)),
                  pl.BlockSpec((1, H), lambda i: (0, 0))],
        out_specs=pl.BlockSpec((_BR, H), lambda i: (i, 0)),
        out_shape=jax.ShapeDtypeStruct((R, H), _f32),
    )(qpart, qpart, W_jk, b_jk.reshape(1, H))


def kernel(x, edge_attr, W_init, b_init, z0_table, We, be, Wg, bg, n_scale,
           n_bias, Wr, br, rn_scale, rn_bias, W_jk, b_jk, edge_index,
           root_index, node_idx, first2second, second2tuple, z0, num_first):
    src = edge_index[0]
    dst = edge_index[1]

    src_p = jnp.pad(src, (0, EP - E))
    dst_p = jnp.pad(dst, (0, EP - E))
    f2s_p = jnp.pad(first2second, (0, EP - E))
    nidx_p = jnp.pad(node_idx, (0, NP - N))
    z0_p = jnp.pad(z0, (0, NP - N))
    root_p = jnp.pad(root_index, (0, RP - R))
    x_p = jnp.pad(x, ((0, NP - N), (0, 0)))
    z0t_p = jnp.pad(z0_table, ((0, H - ZV), (0, 0)))

    embs = [_run_tc_emb(edge_attr, We[l], be[l]) for l in range(L)]

    d2, ri, z0e = _run_sc_prep(f2s_p, second2tuple, nidx_p, root_p, z0_p,
                               z0t_p)

    h = _run_tc_init(x_p, z0e, W_init, b_init)

    for l in range(L):
        part, hg = _run_sc_edge(h, embs[l], src_p, dst_p, d2, ri)
        h = _run_tc_layer(part, h, hg, Wg[l], bg[l], n_scale[l], n_bias[l],
                          Wr[l], br[l], rn_scale[l], rn_bias[l])

    qpart = _run_sc_pool(h, nidx_p)
    return _run_tc_final(qpart, W_jk, b_jk)


# speedup vs baseline: 1.0609x; 1.0609x over previous
import jax
import jax.numpy as jnp
from jax import lax
from jax.experimental import pallas as pl
from jax.experimental.pallas import tpu as pltpu
from jax.experimental.pallas import tpu_sc as plsc

L = 4
H = 128
N = 10000
E = 320000
R = 1000
S2 = 40000
DE = 16
ZV = 100

NC = 2
NS = 16
NW = NC * NS

NP = 10240
EP = 327680
RP = 1024
C = 32
GC = EP // C
PH = 160
NPW = NP // NW
NPT = NP // NS

_f32 = jnp.float32
_i32 = jnp.int32


def _ln(v, scale, bias):
    mu = jnp.mean(v, axis=-1, keepdims=True)
    d = v - mu
    var = jnp.mean(d * d, axis=-1, keepdims=True)
    return d * lax.rsqrt(var + 1e-5) * scale + bias


def _sc_prep(f2s_hbm, s2t_hbm, nidx_hbm, root_hbm, z0_hbm, z0t_hbm,
             d2_hbm, ri_hbm, z0e_hbm,
             s2t_v, root_v, idx_v, out_v, zi_v, zr_v, sem):
    cid = lax.axis_index("c")
    sid = lax.axis_index("s")
    wid = sid * NC + cid

    pltpu.sync_copy(s2t_hbm, s2t_v)
    pltpu.sync_copy(root_hbm, root_v)

    def d2_chunk(i, _):
        base = wid * (EP // NW) + i * 1024

        pltpu.sync_copy(f2s_hbm.at[pl.ds(base, 1024)], idx_v)

        def gat(j, _):
            g = plsc.load_gather(s2t_v, [idx_v[pl.ds(j * 16, 16)]])
            out_v[pl.ds(j * 16, 16)] = g
            return 0

        lax.fori_loop(0, 64, gat, 0)
        pltpu.sync_copy(out_v, d2_hbm.at[pl.ds(base, 1024)])
        return 0

    lax.fori_loop(0, EP // NW // 1024, d2_chunk, 0)

    nbase = wid * NPW
    pltpu.sync_copy(nidx_hbm.at[pl.ds(nbase, NPW)], idx_v.at[pl.ds(0, NPW)])

    def gat_r(j, _):
        g = plsc.load_gather(root_v, [idx_v[pl.ds(j * 16, 16)]])
        out_v[pl.ds(j * 16, 16)] = g
        return 0

    lax.fori_loop(0, NPW // 16, gat_r, 0)
    pltpu.sync_copy(out_v.at[pl.ds(0, NPW)], ri_hbm.at[pl.ds(nbase, NPW)])

    def z0_chunk(j, _):
        b0 = nbase + j * 80
        pltpu.sync_copy(z0_hbm.at[pl.ds(b0, 80)], zi_v)
        pltpu.async_copy(z0t_hbm.at[zi_v], zr_v, sem).wait()
        pltpu.sync_copy(zr_v, z0e_hbm.at[pl.ds(b0, 80)])
        return 0

    lax.fori_loop(0, NPW // 80, z0_chunk, 0)


def _run_sc_prep(f2s_p, s2t, nidx_p, root_p, z0_p, z0t_p):
    mesh = plsc.VectorSubcoreMesh(core_axis_name="c", subcore_axis_name="s")
    fn = pl.kernel(
        _sc_prep,
        out_type=[
            jax.ShapeDtypeStruct((EP,), _i32),
            jax.ShapeDtypeStruct((NP,), _i32),
            jax.ShapeDtypeStruct((NP, H), _f32),
        ],
        mesh=mesh,
        compiler_params=pltpu.CompilerParams(needs_layout_passes=False),
        scratch_types=[
            pltpu.VMEM((S2,), _i32),
            pltpu.VMEM((RP,), _i32),
            pltpu.VMEM((1024,), _i32),
            pltpu.VMEM((1024,), _i32),
            pltpu.VMEM((80,), _i32),
            pltpu.VMEM((80, H), _f32),
            pltpu.SemaphoreType.DMA,
        ],
    )
    return fn(f2s_p, s2t, nidx_p, root_p, z0_p, z0t_p)


def _sc_edge(h_hbm, emb_hbm, src_hbm, dst_hbm, d2_hbm, ri_hbm,
             part_hbm, hg_hbm,
             srcf, dstf, d2f, idx30, idx31, hgi_v, rows0, rows1, emb0, emb1,
             m0, m1, semd0, semd1, sems0, sems1, semz, acc):
    cid = lax.axis_index("c")
    sid = lax.axis_index("s")
    wid = sid * NC + cid

    rows = (rows0, rows1)
    embv = (emb0, emb1)
    mv = (m0, m1)
    idx3 = (idx30, idx31)
    semd = (semd0, semd1)
    sems = (sems0, sems1)

    def zrow(r, _):
        for k in range(H // 16):
            m0[r, pl.ds(k * 16, 16)] = jnp.zeros((16,), _f32)
        return 0

    lax.fori_loop(0, C, zrow, 0)
    for b in range(NPT // C):
        pltpu.async_copy(m0, acc.at[pl.ds(sid * NPT + b * C, C)], semz)
    for b in range(NPT // C):
        pltpu.make_async_copy(m0, acc.at[pl.ds(0, C)], semz).wait()

    htab = h_hbm.at[cid]
    nb = wid * NPW
    nhg = NPW // C
    pltpu.sync_copy(ri_hbm.at[pl.ds(nb, NPW)], srcf.at[pl.ds(0, NPW)])
    hbuf = (emb0, emb1)

    def hg_start(j):
        pltpu.async_copy(htab.at[srcf.at[pl.ds(j * C, C)]], hbuf[j % 2],
                         semd[j % 2])

    hg_start(0)
    hg_start(1)
    for j in range(nhg):
        pltpu.make_async_copy(htab.at[srcf.at[pl.ds(j * C, C)]],
                              hbuf[j % 2], semd[j % 2]).wait()
        pltpu.sync_copy(hbuf[j % 2], hg_hbm.at[pl.ds(nb + j * C, C)])
        if j + 2 < nhg:
            hg_start(j + 2)

    plsc.subcore_barrier()

    def run_phase(cbase):
        ebase = cbase * C

        pltpu.sync_copy(src_hbm.at[pl.ds(ebase, PH * C)], srcf)
        pltpu.sync_copy(dst_hbm.at[pl.ds(ebase, PH * C)], dstf)
        pltpu.sync_copy(d2_hbm.at[pl.ds(ebase, PH * C)], d2f)

        def start_dat(c, s):
            pltpu.async_copy(htab.at[srcf.at[pl.ds(c * C, C)]],
                             rows[s], semd[s])
            pltpu.async_copy(emb_hbm.at[pl.ds(ebase + c * C, C)],
                             embv[s], semd[s])

        def wait_dat(c, s):
            pltpu.make_async_copy(htab.at[srcf.at[pl.ds(c * C, C)]],
                                  rows[s], semd[s]).wait()
            pltpu.make_async_copy(emb_hbm.at[pl.ds(ebase + c * C, C)],
                                  embv[s], semd[s]).wait()

        def stage_idx3(c, s):
            for k in range(C // 16):
                idx3[s][0, pl.ds(k * 16, 16)] = dstf[pl.ds(c * C + k * 16, 16)]
                idx3[s][1, pl.ds(k * 16, 16)] = d2f[pl.ds(c * C + k * 16, 16)]

        def compute(s):
            r_s, e_s, m_s = rows[s], embv[s], mv[s]

            def row(r, _):
                for u in range(2):
                    for k in range(H // 16):
                        sl = pl.ds(k * 16, 16)
                        m_s[2 * r + u, sl] = jnp.maximum(
                            r_s[2 * r + u, sl] + e_s[2 * r + u, sl], 0.0)
                return 0

            lax.fori_loop(0, C // 2, row, 0)

        def start_scat(s):
            pltpu.async_copy(mv[s], acc.at[idx3[s].at[0]], sems[s],
                             add=True)
            pltpu.async_copy(mv[s], acc.at[idx3[s].at[1]], sems[s],
                             add=True)

        def wait_scat(s):
            pltpu.make_async_copy(mv[s], acc.at[idx3[s].at[0]],
                                  sems[s]).wait()
            pltpu.make_async_copy(mv[s], acc.at[idx3[s].at[1]],
                                  sems[s]).wait()

        start_dat(0, 0)
        start_dat(1, 1)
        for s in range(2):
            wait_dat(s, s)
            stage_idx3(s, s)
            compute(s)
            start_scat(s)
            start_dat(s + 2, s)

        def pair(j, _):
            for s in range(2):
                c = 2 * j + s
                wait_dat(c, s)
                wait_scat(s)
                stage_idx3(c, s)
                compute(s)
                start_scat(s)
                start_dat(c + 2, s)
            return 0

        lax.fori_loop(1, PH // 2 - 1, pair, 0)

        for s in range(2):
            c = PH - 2 + s
            wait_dat(c, s)
            wait_scat(s)
            stage_idx3(c, s)
            compute(s)
            start_scat(s)
        wait_scat(0)
        wait_scat(1)

    base0 = wid * (GC // NW)
    for p in range(GC // NW // PH):
        run_phase(base0 + p * PH)

    plsc.subcore_barrier()
    pltpu.sync_copy(acc.at[pl.ds(sid * NPT, NPT)],
                    part_hbm.at[cid, pl.ds(sid * NPT, NPT)])


def _run_sc_edge(h, emb, src_p, dst_p, d2, ri):
    mesh = plsc.VectorSubcoreMesh(core_axis_name="c", subcore_axis_name="s")
    fn = pl.kernel(
        _sc_edge,
        out_type=[
            jax.ShapeDtypeStruct((NC, NP, H), _f32),
            jax.ShapeDtypeStruct((NP, H), _f32),
        ],
        mesh=mesh,
        compiler_params=pltpu.CompilerParams(internal_scratch_in_bytes=4096),
        scratch_types=[
            pltpu.VMEM((PH * C,), _i32),
            pltpu.VMEM((PH * C,), _i32),
            pltpu.VMEM((PH * C,), _i32),
            pltpu.VMEM((2, C), _i32),
            pltpu.VMEM((2, C), _i32),
            pltpu.VMEM((C,), _i32),
            pltpu.VMEM((C, H), _f32),
            pltpu.VMEM((C, H), _f32),
            pltpu.VMEM((C, H), _f32),
            pltpu.VMEM((C, H), _f32),
            pltpu.VMEM((C, H), _f32),
            pltpu.VMEM((C, H), _f32),
            pltpu.SemaphoreType.DMA,
            pltpu.SemaphoreType.DMA,
            pltpu.SemaphoreType.DMA,
            pltpu.SemaphoreType.DMA,
            pltpu.SemaphoreType.DMA,
            pltpu.VMEM_SHARED((NP, H), _f32),
        ],
    )
    return fn(h, emb, src_p, dst_p, d2, ri)


def _sc_pool(h_hbm, nidx_hbm, part_hbm, idx_v, rows_v, acc):
    cid = lax.axis_index("c")
    sid = lax.axis_index("s")
    wid = sid * NC + cid

    def zrow(r, _):
        for k in range(H // 16):
            rows_v[r, pl.ds(k * 16, 16)] = jnp.zeros((16,), _f32)
        return 0

    lax.fori_loop(0, 80, zrow, 0)
    rpt = RP // NS
    pltpu.sync_copy(rows_v.at[pl.ds(0, rpt)],
                    acc.at[pl.ds(sid * rpt, rpt)])
    plsc.subcore_barrier()

    def chunk(j, _):
        b0 = wid * NPW + j * 80
        pltpu.sync_copy(nidx_hbm.at[pl.ds(b0, 80)], idx_v)
        pltpu.sync_copy(h_hbm.at[cid, pl.ds(b0, 80)], rows_v)
        pltpu.sync_copy(rows_v, acc.at[idx_v], add=True)
        return 0

    lax.fori_loop(0, NPW // 80, chunk, 0)

    plsc.subcore_barrier()
    pltpu.sync_copy(acc.at[pl.ds(sid * rpt, rpt)],
                    part_hbm.at[cid, pl.ds(sid * rpt, rpt)])


def _run_sc_pool(h, nidx_p):
    mesh = plsc.VectorSubcoreMesh(core_axis_name="c", subcore_axis_name="s")
    fn = pl.kernel(
        _sc_pool,
        out_type=jax.ShapeDtypeStruct((NC, RP, H), _f32),
        mesh=mesh,
        scratch_types=[
            pltpu.VMEM((80,), _i32),
            pltpu.VMEM((80, H), _f32),
            pltpu.VMEM_SHARED((RP, H), _f32),
        ],
    )
    return fn(h, nidx_p)


_BN = 1024


def _tc_init_body(x_ref, z0e_ref, Wi_ref, bi_ref, o_ref):
    pid = pl.program_id(0)
    v = jnp.dot(x_ref[...], Wi_ref[...], preferred_element_type=_f32)
    v = v + bi_ref[...] + z0e_ref[...]
    rows = pid * _BN + lax.broadcasted_iota(_i32, (_BN, H), 0)
    v = jnp.where(rows < N, v, 0.0)
    o_ref[0] = v
    o_ref[1] = v


def _run_tc_init(x_p, z0e, W_init, b_init):
    return pl.pallas_call(
        _tc_init_body,
        grid=(NP // _BN,),
        in_specs=[
            pl.BlockSpec((_BN, H), lambda i: (i, 0)),
            pl.BlockSpec((_BN, H), lambda i: (i, 0)),
            pl.BlockSpec((H, H), lambda i: (0, 0)),
            pl.BlockSpec((1, H), lambda i: (0, 0)),
        ],
        out_specs=pl.BlockSpec((NC, _BN, H), lambda i: (0, i, 0)),
        out_shape=jax.ShapeDtypeStruct((NC, NP, H), _f32),
    )(x_p, z0e, W_init, b_init.reshape(1, H))


_BE = 2048


def _tc_emb_body(ea_ref, We_ref, be_ref, o_ref):
    pid = pl.program_id(0)
    v = jnp.dot(ea_ref[...], We_ref[...], preferred_element_type=_f32)
    v = v + be_ref[...]
    rows = pid * _BE + lax.broadcasted_iota(_i32, (_BE, H), 0)
    o_ref[...] = jnp.where(rows < E, v, -1e30)


def _run_tc_emb(ea_p, We_l, be_l):
    return pl.pallas_call(
        _tc_emb_body,
        grid=(EP // _BE,),
        in_specs=[
            pl.BlockSpec((_BE, DE), lambda i: (i, 0)),
            pl.BlockSpec((DE, H), lambda i: (0, 0)),
            pl.BlockSpec((1, H), lambda i: (0, 0)),
        ],
        out_specs=pl.BlockSpec((_BE, H), lambda i: (i, 0)),
        out_shape=jax.ShapeDtypeStruct((EP, H), _f32),
    )(ea_p, We_l, be_l.reshape(1, H))


def _tc_layer_body(p0_ref, p1_ref, h_ref, hg_ref, Wg_ref, bg_ref, ns_ref,
                   nb_ref, Wr_ref, br_ref, rs_ref, rb_ref, o_ref):
    pid = pl.program_id(0)
    agg = p0_ref[0] + p1_ref[0]
    out = jnp.dot(agg, Wg_ref[...], preferred_element_type=_f32) + bg_ref[...]
    out = _ln(out, ns_ref[...], nb_ref[...])
    hr = h_ref[0] + hg_ref[...]
    hr = jnp.dot(hr, Wr_ref[...], preferred_element_type=_f32) + br_ref[...]
    hr = jnp.maximum(hr, 0.0)
    hr = _ln(hr, rs_ref[...], rb_ref[...])
    v = out + hr
    rows = pid * _BN + lax.broadcasted_iota(_i32, (_BN, H), 0)
    v = jnp.where(rows < N, v, 0.0)
    o_ref[0] = v
    o_ref[1] = v


def _run_tc_layer(part, h, hg, Wg_l, bg_l, ns_l, nb_l, Wr_l, br_l, rs_l,
                  rb_l):
    row_spec = pl.BlockSpec((_BN, H), lambda i: (i, 0))
    p0 = pl.BlockSpec((1, _BN, H), lambda i: (0, i, 0))
    p1 = pl.BlockSpec((1, _BN, H), lambda i: (1, i, 0))
    mat_spec = pl.BlockSpec((H, H), lambda i: (0, 0))
    vec_spec = pl.BlockSpec((1, H), lambda i: (0, 0))
    return pl.pallas_call(
        _tc_layer_body,
        grid=(NP // _BN,),
        in_specs=[p0, p1, p0, row_spec,
                  mat_spec, vec_spec, vec_spec, vec_spec,
                  mat_spec, vec_spec, vec_spec, vec_spec],
        out_specs=pl.BlockSpec((NC, _BN, H), lambda i: (0, i, 0)),
        out_shape=jax.ShapeDtypeStruct((NC, NP, H), _f32),
    )(part, part, h, hg, Wg_l, bg_l.reshape(1, H), ns_l.reshape(1, H),
      nb_l.reshape(1, H), Wr_l, br_l.reshape(1, H), rs_l.reshape(1, H),
      rb_l.reshape(1, H))


_BR = 200


def _tc_final_body(q0_ref, q1_ref, Wj_ref, bj_ref, o_ref):
    q = q0_ref[0] + q1_ref[0]
    o_ref[...] = jnp.dot(q, Wj_ref[...], preferred_element_type=_f32) + bj_ref[...]


def _run_tc_final(qpart, W_jk, b_jk):
    return pl.pallas_call(
        _tc_final_body,
        grid=(R // _BR,),
        in_specs=[pl.BlockSpec((1, _BR, H), lambda i: (0, i, 0)),
                  pl.BlockSpec((1, _BR, H), lambda i: (1, i, 0)),
                  pl.BlockSpec((H, H), lambda i: (0, 0)),
                  pl.BlockSpec((1, H), lambda i: (0, 0))],
        out_specs=pl.BlockSpec((_BR, H), lambda i: (i, 0)),
        out_shape=jax.ShapeDtypeStruct((R, H), _f32),
    )(qpart, qpart, W_jk, b_jk.reshape(1, H))


def kernel(x, edge_attr, W_init, b_init, z0_table, We, be, Wg, bg, n_scale,
           n_bias, Wr, br, rn_scale, rn_bias, W_jk, b_jk, edge_index,
           root_index, node_idx, first2second, second2tuple, z0, num_first):
    src = edge_index[0]
    dst = edge_index[1]

    src_p = jnp.pad(src, (0, EP - E))
    dst_p = jnp.pad(dst, (0, EP - E))
    f2s_p = jnp.pad(first2second, (0, EP - E))
    nidx_p = jnp.pad(node_idx, (0, NP - N))
    z0_p = jnp.pad(z0, (0, NP - N))
    root_p = jnp.pad(root_index, (0, RP - R))
    x_p = jnp.pad(x, ((0, NP - N), (0, 0)))
    ea_p = jnp.pad(edge_attr, ((0, EP - E), (0, 0)))
    z0t_p = jnp.pad(z0_table, ((0, H - ZV), (0, 0)))

    embs = [_run_tc_emb(ea_p, We[l], be[l]) for l in range(L)]

    d2, ri, z0e = _run_sc_prep(f2s_p, second2tuple, nidx_p, root_p, z0_p,
                               z0t_p)

    h = _run_tc_init(x_p, z0e, W_init, b_init)

    for l in range(L):
        part, hg = _run_sc_edge(h, embs[l], src_p, dst_p, d2, ri)
        h = _run_tc_layer(part, h, hg, Wg[l], bg[l], n_scale[l], n_bias[l],
                          Wr[l], br[l], rn_scale[l], rn_bias[l])

    qpart = _run_sc_pool(h, nidx_p)
    return _run_tc_final(qpart, W_jk, b_jk)


# speedup vs baseline: 1.0821x; 1.0200x over previous
import jax
import jax.numpy as jnp
from jax import lax
from jax.experimental import pallas as pl
from jax.experimental.pallas import tpu as pltpu
from jax.experimental.pallas import tpu_sc as plsc

L = 4
H = 128
N = 10000
E = 320000
R = 1000
S2 = 40000
DE = 16
ZV = 100

NC = 2
NS = 16
NW = NC * NS

NP = 10240
EP = 327680
RP = 1024
C = 32
GC = EP // C
PH = 160
NPW = NP // NW
NPT = NP // NS

_f32 = jnp.float32
_i32 = jnp.int32


def _ln(v, scale, bias):
    mu = jnp.mean(v, axis=-1, keepdims=True)
    d = v - mu
    var = jnp.mean(d * d, axis=-1, keepdims=True)
    return d * lax.rsqrt(var + 1e-5) * scale + bias


def _sc_prep(f2s_hbm, s2t_hbm, nidx_hbm, root_hbm, z0_hbm, z0t_hbm,
             d2_hbm, ri_hbm, z0e_hbm,
             s2t_v, root_v, idx_v, out_v, zi_v, zr_v, sem):
    cid = lax.axis_index("c")
    sid = lax.axis_index("s")
    wid = sid * NC + cid

    pltpu.sync_copy(s2t_hbm, s2t_v)
    pltpu.sync_copy(root_hbm, root_v)

    def d2_chunk(i, _):
        base = wid * (EP // NW) + i * 1024

        pltpu.sync_copy(f2s_hbm.at[pl.ds(base, 1024)], idx_v)

        def gat(j, _):
            g = plsc.load_gather(s2t_v, [idx_v[pl.ds(j * 16, 16)]])
            out_v[pl.ds(j * 16, 16)] = g
            return 0

        lax.fori_loop(0, 64, gat, 0)
        pltpu.sync_copy(out_v, d2_hbm.at[pl.ds(base, 1024)])
        return 0

    lax.fori_loop(0, EP // NW // 1024, d2_chunk, 0)

    nbase = wid * NPW
    pltpu.sync_copy(nidx_hbm.at[pl.ds(nbase, NPW)], idx_v.at[pl.ds(0, NPW)])

    def gat_r(j, _):
        g = plsc.load_gather(root_v, [idx_v[pl.ds(j * 16, 16)]])
        out_v[pl.ds(j * 16, 16)] = g
        return 0

    lax.fori_loop(0, NPW // 16, gat_r, 0)
    pltpu.sync_copy(out_v.at[pl.ds(0, NPW)], ri_hbm.at[pl.ds(nbase, NPW)])

    def z0_chunk(j, _):
        b0 = nbase + j * 80
        pltpu.sync_copy(z0_hbm.at[pl.ds(b0, 80)], zi_v)
        pltpu.async_copy(z0t_hbm.at[zi_v], zr_v, sem).wait()
        pltpu.sync_copy(zr_v, z0e_hbm.at[pl.ds(b0, 80)])
        return 0

    lax.fori_loop(0, NPW // 80, z0_chunk, 0)


def _run_sc_prep(f2s_p, s2t, nidx_p, root_p, z0_p, z0t_p):
    mesh = plsc.VectorSubcoreMesh(core_axis_name="c", subcore_axis_name="s")
    fn = pl.kernel(
        _sc_prep,
        out_type=[
            jax.ShapeDtypeStruct((EP,), _i32),
            jax.ShapeDtypeStruct((NP,), _i32),
            jax.ShapeDtypeStruct((NP, H), _f32),
        ],
        mesh=mesh,
        compiler_params=pltpu.CompilerParams(needs_layout_passes=False),
        scratch_types=[
            pltpu.VMEM((S2,), _i32),
            pltpu.VMEM((RP,), _i32),
            pltpu.VMEM((1024,), _i32),
            pltpu.VMEM((1024,), _i32),
            pltpu.VMEM((80,), _i32),
            pltpu.VMEM((80, H), _f32),
            pltpu.SemaphoreType.DMA,
        ],
    )
    return fn(f2s_p, s2t, nidx_p, root_p, z0_p, z0t_p)


def _sc_edge(h_hbm, emb_hbm, src_hbm, dst_hbm, d2_hbm, ri_hbm,
             part_hbm, hg_hbm,
             srcf, dstf, d2f, idx30, idx31, hgi_v, rows0, rows1, emb0, emb1,
             m0, m1, semd0, semd1, sems0, sems1, semz, acc):
    cid = lax.axis_index("c")
    sid = lax.axis_index("s")
    wid = sid * NC + cid

    rows = (rows0, rows1)
    embv = (emb0, emb1)
    mv = (m0, m1)
    idx3 = (idx30, idx31)
    semd = (semd0, semd1)
    sems = (sems0, sems1)

    def zrow(r, _):
        for k in range(H // 16):
            m0[r, pl.ds(k * 16, 16)] = jnp.zeros((16,), _f32)
        return 0

    lax.fori_loop(0, C, zrow, 0)
    for b in range(NPT // C):
        pltpu.async_copy(m0, acc.at[pl.ds(sid * NPT + b * C, C)], semz)
    for b in range(NPT // C):
        pltpu.make_async_copy(m0, acc.at[pl.ds(0, C)], semz).wait()

    htab = h_hbm
    nb = wid * NPW
    nhg = NPW // C
    pltpu.sync_copy(ri_hbm.at[pl.ds(nb, NPW)], srcf.at[pl.ds(0, NPW)])
    hbuf = (emb0, emb1)

    def hg_start(j):
        pltpu.async_copy(htab.at[srcf.at[pl.ds(j * C, C)]], hbuf[j % 2],
                         semd[j % 2])

    hg_start(0)
    hg_start(1)
    for j in range(nhg):
        pltpu.make_async_copy(htab.at[srcf.at[pl.ds(j * C, C)]],
                              hbuf[j % 2], semd[j % 2]).wait()
        pltpu.sync_copy(hbuf[j % 2], hg_hbm.at[pl.ds(nb + j * C, C)])
        if j + 2 < nhg:
            hg_start(j + 2)

    plsc.subcore_barrier()

    def run_phase(cbase):
        ebase = cbase * C

        pltpu.sync_copy(src_hbm.at[pl.ds(ebase, PH * C)], srcf)
        pltpu.sync_copy(dst_hbm.at[pl.ds(ebase, PH * C)], dstf)
        pltpu.sync_copy(d2_hbm.at[pl.ds(ebase, PH * C)], d2f)

        def start_dat(c, s):
            pltpu.async_copy(htab.at[srcf.at[pl.ds(c * C, C)]],
                             rows[s], semd[s])
            pltpu.async_copy(emb_hbm.at[pl.ds(ebase + c * C, C)],
                             embv[s], semd[s])

        def wait_dat(c, s):
            pltpu.make_async_copy(htab.at[srcf.at[pl.ds(c * C, C)]],
                                  rows[s], semd[s]).wait()
            pltpu.make_async_copy(emb_hbm.at[pl.ds(ebase + c * C, C)],
                                  embv[s], semd[s]).wait()

        def stage_idx3(c, s):
            for k in range(C // 16):
                idx3[s][0, pl.ds(k * 16, 16)] = dstf[pl.ds(c * C + k * 16, 16)]
                idx3[s][1, pl.ds(k * 16, 16)] = d2f[pl.ds(c * C + k * 16, 16)]

        def compute(s):
            r_s, e_s, m_s = rows[s], embv[s], mv[s]

            def row(r, _):
                for u in range(2):
                    for k in range(H // 16):
                        sl = pl.ds(k * 16, 16)
                        m_s[2 * r + u, sl] = jnp.maximum(
                            r_s[2 * r + u, sl] + e_s[2 * r + u, sl], 0.0)
                return 0

            lax.fori_loop(0, C // 2, row, 0)

        def start_scat(s):
            pltpu.async_copy(mv[s], acc.at[idx3[s].at[0]], sems[s],
                             add=True)
            pltpu.async_copy(mv[s], acc.at[idx3[s].at[1]], sems[s],
                             add=True)

        def wait_scat(s):
            pltpu.make_async_copy(mv[s], acc.at[idx3[s].at[0]],
                                  sems[s]).wait()
            pltpu.make_async_copy(mv[s], acc.at[idx3[s].at[1]],
                                  sems[s]).wait()

        start_dat(0, 0)
        start_dat(1, 1)
        for s in range(2):
            wait_dat(s, s)
            stage_idx3(s, s)
            compute(s)
            start_scat(s)
            start_dat(s + 2, s)

        def pair(j, _):
            for s in range(2):
                c = 2 * j + s
                wait_dat(c, s)
                wait_scat(s)
                stage_idx3(c, s)
                compute(s)
                start_scat(s)
                start_dat(c + 2, s)
            return 0

        lax.fori_loop(1, PH // 2 - 1, pair, 0)

        for s in range(2):
            c = PH - 2 + s
            wait_dat(c, s)
            wait_scat(s)
            stage_idx3(c, s)
            compute(s)
            start_scat(s)
        wait_scat(0)
        wait_scat(1)

    base0 = wid * (GC // NW)
    for p in range(GC // NW // PH):
        run_phase(base0 + p * PH)

    plsc.subcore_barrier()
    pltpu.sync_copy(acc.at[pl.ds(sid * NPT, NPT)],
                    part_hbm.at[cid, pl.ds(sid * NPT, NPT)])


def _run_sc_edge(h, emb, src_p, dst_p, d2, ri):
    mesh = plsc.VectorSubcoreMesh(core_axis_name="c", subcore_axis_name="s")
    fn = pl.kernel(
        _sc_edge,
        out_type=[
            jax.ShapeDtypeStruct((NC, NP, H), _f32),
            jax.ShapeDtypeStruct((NP, H), _f32),
        ],
        mesh=mesh,
        compiler_params=pltpu.CompilerParams(internal_scratch_in_bytes=4096),
        scratch_types=[
            pltpu.VMEM((PH * C,), _i32),
            pltpu.VMEM((PH * C,), _i32),
            pltpu.VMEM((PH * C,), _i32),
            pltpu.VMEM((2, C), _i32),
            pltpu.VMEM((2, C), _i32),
            pltpu.VMEM((C,), _i32),
            pltpu.VMEM((C, H), _f32),
            pltpu.VMEM((C, H), _f32),
            pltpu.VMEM((C, H), _f32),
            pltpu.VMEM((C, H), _f32),
            pltpu.VMEM((C, H), _f32),
            pltpu.VMEM((C, H), _f32),
            pltpu.SemaphoreType.DMA,
            pltpu.SemaphoreType.DMA,
            pltpu.SemaphoreType.DMA,
            pltpu.SemaphoreType.DMA,
            pltpu.SemaphoreType.DMA,
            pltpu.VMEM_SHARED((NP, H), _f32),
        ],
    )
    return fn(h, emb, src_p, dst_p, d2, ri)


def _sc_pool(h_hbm, nidx_hbm, part_hbm, idx_v, rows_v, acc):
    cid = lax.axis_index("c")
    sid = lax.axis_index("s")
    wid = sid * NC + cid

    def zrow(r, _):
        for k in range(H // 16):
            rows_v[r, pl.ds(k * 16, 16)] = jnp.zeros((16,), _f32)
        return 0

    lax.fori_loop(0, 80, zrow, 0)
    rpt = RP // NS
    pltpu.sync_copy(rows_v.at[pl.ds(0, rpt)],
                    acc.at[pl.ds(sid * rpt, rpt)])
    plsc.subcore_barrier()

    def chunk(j, _):
        b0 = wid * NPW + j * 80
        pltpu.sync_copy(nidx_hbm.at[pl.ds(b0, 80)], idx_v)
        pltpu.sync_copy(h_hbm.at[pl.ds(b0, 80)], rows_v)
        pltpu.sync_copy(rows_v, acc.at[idx_v], add=True)
        return 0

    lax.fori_loop(0, NPW // 80, chunk, 0)

    plsc.subcore_barrier()
    pltpu.sync_copy(acc.at[pl.ds(sid * rpt, rpt)],
                    part_hbm.at[cid, pl.ds(sid * rpt, rpt)])


def _run_sc_pool(h, nidx_p):
    mesh = plsc.VectorSubcoreMesh(core_axis_name="c", subcore_axis_name="s")
    fn = pl.kernel(
        _sc_pool,
        out_type=jax.ShapeDtypeStruct((NC, RP, H), _f32),
        mesh=mesh,
        scratch_types=[
            pltpu.VMEM((80,), _i32),
            pltpu.VMEM((80, H), _f32),
            pltpu.VMEM_SHARED((RP, H), _f32),
        ],
    )
    return fn(h, nidx_p)


_BN = 1024


def _tc_init_body(x_ref, z0e_ref, Wi_ref, bi_ref, o_ref):
    pid = pl.program_id(0)
    v = jnp.dot(x_ref[...], Wi_ref[...], preferred_element_type=_f32)
    v = v + bi_ref[...] + z0e_ref[...]
    rows = pid * _BN + lax.broadcasted_iota(_i32, (_BN, H), 0)
    o_ref[...] = jnp.where(rows < N, v, 0.0)


def _run_tc_init(x_p, z0e, W_init, b_init):
    return pl.pallas_call(
        _tc_init_body,
        grid=(NP // _BN,),
        in_specs=[
            pl.BlockSpec((_BN, H), lambda i: (i, 0)),
            pl.BlockSpec((_BN, H), lambda i: (i, 0)),
            pl.BlockSpec((H, H), lambda i: (0, 0)),
            pl.BlockSpec((1, H), lambda i: (0, 0)),
        ],
        out_specs=pl.BlockSpec((_BN, H), lambda i: (i, 0)),
        out_shape=jax.ShapeDtypeStruct((NP, H), _f32),
    )(x_p, z0e, W_init, b_init.reshape(1, H))


_BE = 2048


def _tc_emb_body(ea_ref, We_ref, be_ref, o_ref):
    pid = pl.program_id(0)
    v = jnp.dot(ea_ref[...], We_ref[...], preferred_element_type=_f32)
    v = v + be_ref[...]
    rows = pid * _BE + lax.broadcasted_iota(_i32, (_BE, H), 0)
    o_ref[...] = jnp.where(rows < E, v, -1e30)


def _run_tc_emb(ea_p, We_l, be_l):
    return pl.pallas_call(
        _tc_emb_body,
        grid=(EP // _BE,),
        in_specs=[
            pl.BlockSpec((_BE, DE), lambda i: (i, 0)),
            pl.BlockSpec((DE, H), lambda i: (0, 0)),
            pl.BlockSpec((1, H), lambda i: (0, 0)),
        ],
        out_specs=pl.BlockSpec((_BE, H), lambda i: (i, 0)),
        out_shape=jax.ShapeDtypeStruct((EP, H), _f32),
    )(ea_p, We_l, be_l.reshape(1, H))


def _tc_layer_body(p0_ref, p1_ref, h_ref, hg_ref, Wg_ref, bg_ref, ns_ref,
                   nb_ref, Wr_ref, br_ref, rs_ref, rb_ref, o_ref):
    pid = pl.program_id(0)
    agg = p0_ref[0] + p1_ref[0]
    out = jnp.dot(agg, Wg_ref[...], preferred_element_type=_f32) + bg_ref[...]
    out = _ln(out, ns_ref[...], nb_ref[...])
    hr = h_ref[...] + hg_ref[...]
    hr = jnp.dot(hr, Wr_ref[...], preferred_element_type=_f32) + br_ref[...]
    hr = jnp.maximum(hr, 0.0)
    hr = _ln(hr, rs_ref[...], rb_ref[...])
    v = out + hr
    rows = pid * _BN + lax.broadcasted_iota(_i32, (_BN, H), 0)
    o_ref[...] = jnp.where(rows < N, v, 0.0)


def _run_tc_layer(part, h, hg, Wg_l, bg_l, ns_l, nb_l, Wr_l, br_l, rs_l,
                  rb_l):
    row_spec = pl.BlockSpec((_BN, H), lambda i: (i, 0))
    p0 = pl.BlockSpec((1, _BN, H), lambda i: (0, i, 0))
    p1 = pl.BlockSpec((1, _BN, H), lambda i: (1, i, 0))
    mat_spec = pl.BlockSpec((H, H), lambda i: (0, 0))
    vec_spec = pl.BlockSpec((1, H), lambda i: (0, 0))
    return pl.pallas_call(
        _tc_layer_body,
        grid=(NP // _BN,),
        in_specs=[p0, p1, row_spec, row_spec,
                  mat_spec, vec_spec, vec_spec, vec_spec,
                  mat_spec, vec_spec, vec_spec, vec_spec],
        out_specs=row_spec,
        out_shape=jax.ShapeDtypeStruct((NP, H), _f32),
    )(part, part, h, hg, Wg_l, bg_l.reshape(1, H), ns_l.reshape(1, H),
      nb_l.reshape(1, H), Wr_l, br_l.reshape(1, H), rs_l.reshape(1, H),
      rb_l.reshape(1, H))


_BR = 200


def _tc_final_body(q0_ref, q1_ref, Wj_ref, bj_ref, o_ref):
    q = q0_ref[0] + q1_ref[0]
    o_ref[...] = jnp.dot(q, Wj_ref[...], preferred_element_type=_f32) + bj_ref[...]


def _run_tc_final(qpart, W_jk, b_jk):
    return pl.pallas_call(
        _tc_final_body,
        grid=(R // _BR,),
        in_specs=[pl.BlockSpec((1, _BR, H), lambda i: (0, i, 0)),
                  pl.BlockSpec((1, _BR, H), lambda i: (1, i, 0)),
                  pl.BlockSpec((H, H), lambda i: (0, 0)),
                  pl.BlockSpec((1, H), lambda i: (0, 0))],
        out_specs=pl.BlockSpec((_BR, H), lambda i: (i, 0)),
        out_shape=jax.ShapeDtypeStruct((R, H), _f32),
    )(qpart, qpart, W_jk, b_jk.reshape(1, H))


def kernel(x, edge_attr, W_init, b_init, z0_table, We, be, Wg, bg, n_scale,
           n_bias, Wr, br, rn_scale, rn_bias, W_jk, b_jk, edge_index,
           root_index, node_idx, first2second, second2tuple, z0, num_first):
    src = edge_index[0]
    dst = edge_index[1]

    src_p = jnp.pad(src, (0, EP - E))
    dst_p = jnp.pad(dst, (0, EP - E))
    f2s_p = jnp.pad(first2second, (0, EP - E))
    nidx_p = jnp.pad(node_idx, (0, NP - N))
    z0_p = jnp.pad(z0, (0, NP - N))
    root_p = jnp.pad(root_index, (0, RP - R))
    x_p = jnp.pad(x, ((0, NP - N), (0, 0)))
    ea_p = jnp.pad(edge_attr, ((0, EP - E), (0, 0)))
    z0t_p = jnp.pad(z0_table, ((0, H - ZV), (0, 0)))

    embs = [_run_tc_emb(ea_p, We[l], be[l]) for l in range(L)]

    d2, ri, z0e = _run_sc_prep(f2s_p, second2tuple, nidx_p, root_p, z0_p,
                               z0t_p)

    h = _run_tc_init(x_p, z0e, W_init, b_init)

    for l in range(L):
        part, hg = _run_sc_edge(h, embs[l], src_p, dst_p, d2, ri)
        h = _run_tc_layer(part, h, hg, Wg[l], bg[l], n_scale[l], n_bias[l],
                          Wr[l], br[l], rn_scale[l], rn_bias[l])

    qpart = _run_sc_pool(h, nidx_p)
    return _run_tc_final(qpart, W_jk, b_jk)
